# baseline (speedup 1.0000x reference)
import functools

import jax
import jax.numpy as jnp
import numpy as np
from jax import lax
from jax.experimental import pallas as pl
from jax.experimental.pallas import tpu as pltpu
from jax.experimental.pallas import tpu_sc as plsc

_LIST_OF_N = [100003, 100019, 100043, 100049]
_D = 128
_DM = 2048
_K = 4
_B, _L, _H = 4, 4096, 4
_HD = _H * _D
_BL = _B * _L
_HBL = _BL // 2
_NW = 32
_WPH = _NW // _H
_CH = 128
_PPW = _H * _HBL // _NW
_NCH = _PPW // _CH

_TL = 2048
_NJ = _HBL // _TL
_BPB = _L // _TL


def _gather_pack_sc(idsA, idsB, table):
  mesh = plsc.VectorSubcoreMesh(core_axis_name="c", subcore_axis_name="s")

  @functools.partial(
      pl.kernel,
      mesh=mesh,
      out_type=jax.ShapeDtypeStruct((_H, _HBL, _D), jnp.int32),
      scratch_types=[
          pltpu.VMEM((_NCH, _CH), jnp.int32),
          pltpu.VMEM((_NCH, _CH), jnp.int32),
          pltpu.VMEM((_CH, _D), jnp.int32),
          pltpu.VMEM((_CH, _D), jnp.int32),
          pltpu.VMEM((_CH, _D), jnp.int32),
          pltpu.VMEM((_CH, _D), jnp.int32),
          pltpu.VMEM((_CH, _D), jnp.int32),
          pltpu.VMEM((_CH, _D), jnp.int32),
          pltpu.SemaphoreType.DMA,
          pltpu.SemaphoreType.DMA,
          pltpu.SemaphoreType.DMA,
          pltpu.SemaphoreType.DMA,
          pltpu.SemaphoreType.DMA,
          pltpu.SemaphoreType.DMA,
      ],
  )
  def k(idsA_hbm, idsB_hbm, table_hbm, out_hbm, idxA, idxB,
        a0, a1, b0, b1, p0, p1, ga0, ga1, gb0, gb1, w0, w1):
    wid = lax.axis_index("s") * 2 + lax.axis_index("c")
    h = wid // _WPH
    base = (wid % _WPH) * _PPW
    pltpu.sync_copy(idsA_hbm.at[wid], idxA)
    pltpu.sync_copy(idsB_hbm.at[wid], idxB)
    abufs = (a0, a1)
    bbufs = (b0, b1)
    pbufs = (p0, p1)
    gasem = (ga0, ga1)
    gbsem = (gb0, gb1)
    wsem = (w0, w1)
    ga = [None] * 2
    gb = [None] * 2
    w = [None] * 2

    def _pack_chunk(ab, bb, pb):
      rnd = jnp.int32(0x8000)
      himask = jnp.int32(-65536)

      def body(r, carry):
        for grp in range(_D // 16):
          sl = pl.ds(grp * 16, 16)
          va = ab[r, sl]
          vb = bb[r, sl]
          lo = lax.shift_right_logical(va + rnd, 16)
          hi = (vb + rnd) & himask
          pb[r, sl] = lo | hi
        return carry
      lax.fori_loop(0, _CH, body, 0)

    ga[0] = pltpu.async_copy(table_hbm.at[idxA.at[0]], a0, ga0)
    gb[0] = pltpu.async_copy(table_hbm.at[idxB.at[0]], b0, gb0)
    for c in range(_NCH):
      s = c % 2
      if c + 1 < _NCH:
        o = (c + 1) % 2
        ga[o] = pltpu.async_copy(table_hbm.at[idxA.at[c + 1]], abufs[o],
                                 gasem[o])
        gb[o] = pltpu.async_copy(table_hbm.at[idxB.at[c + 1]], bbufs[o],
                                 gbsem[o])
      ga[s].wait()
      gb[s].wait()
      if c >= 2:
        w[s].wait()
      _pack_chunk(abufs[s], bbufs[s], pbufs[s])
      w[s] = pltpu.async_copy(
          pbufs[s], out_hbm.at[h, pl.ds(base + c * _CH, _CH)], wsem[s])
    w[0].wait()
    w[1].wait()

  return k(idsA, idsB, table)


def _tc_body(xw_ref, cw_ref, w_ref, out_ref, carry0, carry1):
  j = pl.program_id(0)
  t = pl.program_id(1)

  xw = xw_ref[...]
  xi = jnp.where(t == 0, xw << 16, xw & jnp.int32(-65536))
  x4 = lax.bitcast_convert_type(xi, jnp.float32)
  prev = jnp.where(t == 0, carry0[...], carry1[...])
  prev = jnp.where(j % _BPB == 0, jnp.zeros_like(prev), prev)
  cw = cw_ref[...]
  conv = x4 * cw[:, _K - 1, :][:, None, :]
  for s in range(1, _K):
    shifted = jnp.concatenate(
        [prev[:, 8 - s:, :], x4[:, :_TL - s, :]], axis=1)
    conv = conv + shifted * cw[:, _K - 1 - s, :][:, None, :]
  tail = x4[:, _TL - 8:, :]

  @pl.when(t == 0)
  def _():
    carry0[...] = tail

  @pl.when(t != 0)
  def _():
    carry1[...] = tail

  y4 = (conv * jax.nn.sigmoid(conv) * x4).astype(jnp.bfloat16)
  y = jnp.concatenate([y4[0], y4[1], y4[2], y4[3]], axis=1)
  out_ref[...] = jnp.dot(y, w_ref[...], preferred_element_type=jnp.float32)


def _tc_call(xw, cw4, w_bf16):
  return pl.pallas_call(
      _tc_body,
      grid=(_NJ, 2),
      in_specs=[
          pl.BlockSpec((_H, _TL, _D), lambda j, t: (0, j, 0)),
          pl.BlockSpec((_H, _K, _D), lambda j, t: (0, 0, 0)),
          pl.BlockSpec((_HD, _DM), lambda j, t: (0, 0)),
      ],
      out_specs=pl.BlockSpec((_TL, _DM), lambda j, t: (j + t * _NJ, 0)),
      out_shape=jax.ShapeDtypeStruct((_BL, _DM), jnp.float32),
      scratch_shapes=[
          pltpu.VMEM((_H, 8, _D), jnp.float32),
          pltpu.VMEM((_H, 8, _D), jnp.float32),
      ],
      compiler_params=pltpu.CompilerParams(
          dimension_semantics=("arbitrary", "arbitrary")),
  )(xw, cw4, w_bf16)


def kernel(input_ids, emb_table, conv_w, w_out):
  offsets = jnp.array(np.cumsum([0] + _LIST_OF_N[:-1]), dtype=input_ids.dtype)
  shifted = (input_ids + offsets[None, None, :]).transpose(2, 0, 1)
  shifted = shifted.reshape(_H, _BL)
  idsA = shifted[:, :_HBL].reshape(_NW, _NCH, _CH)
  idsB = shifted[:, _HBL:].reshape(_NW, _NCH, _CH)
  cw4 = conv_w.reshape(_K, _H, _D).transpose(1, 0, 2)
  table_i32 = lax.bitcast_convert_type(emb_table, jnp.int32)
  xw = _gather_pack_sc(idsA, idsB, table_i32)
  out = _tc_call(xw, cw4, w_out.astype(jnp.bfloat16))
  return out.reshape(_B, _L, _DM)


# speedup vs baseline: 2.3478x; 2.3478x over previous
import functools

import jax
import jax.numpy as jnp
import numpy as np
from jax import lax
from jax.experimental import pallas as pl
from jax.experimental.pallas import tpu as pltpu
from jax.experimental.pallas import tpu_sc as plsc

_LIST_OF_N = [100003, 100019, 100043, 100049]
_D = 128
_DM = 2048
_K = 4
_B, _L, _H = 4, 4096, 4
_HD = _H * _D
_BL = _B * _L
_NW = 32
_WPH = _NW // _H
_CH = 128
_NBUF = 6

_TL = 2048
_BPB = _L // _TL
_SPLIT = 1


def _gather_sc(ids3, table, bl):
  rpw = _H * bl // _NW
  nch = rpw // _CH
  mesh = plsc.VectorSubcoreMesh(core_axis_name="c", subcore_axis_name="s")

  @functools.partial(
      pl.kernel,
      mesh=mesh,
      out_type=jax.ShapeDtypeStruct((_H, bl, _D), jnp.float32),
      scratch_types=[
          pltpu.VMEM((nch, _CH), jnp.int32),
      ] + [pltpu.VMEM((_CH, _D), jnp.float32)] * _NBUF
        + [pltpu.SemaphoreType.DMA] * (2 * _NBUF),
  )
  def k(ids_hbm, table_hbm, out_hbm, idx_v, *bufs_sems):
    bufs = bufs_sems[:_NBUF]
    gsem = bufs_sems[_NBUF:2 * _NBUF]
    wsem = bufs_sems[2 * _NBUF:]
    wid = lax.axis_index("s") * 2 + lax.axis_index("c")
    h = wid // _WPH
    base = (wid % _WPH) * rpw
    pltpu.sync_copy(ids_hbm.at[wid], idx_v)
    lead = _NBUF - 2
    g = [None] * _NBUF
    w = [None] * _NBUF
    for c in range(min(_NBUF, nch)):
      g[c] = pltpu.async_copy(table_hbm.at[idx_v.at[c]], bufs[c], gsem[c])
    for c in range(nch):
      s = c % _NBUF
      if _NBUF - lead <= c and c + lead < nch:
        ps = (c + lead) % _NBUF
        w[ps].wait()
        g[ps] = pltpu.async_copy(
            table_hbm.at[idx_v.at[c + lead]], bufs[ps], gsem[ps])
      g[s].wait()
      w[s] = pltpu.async_copy(
          bufs[s], out_hbm.at[h, pl.ds(base + c * _CH, _CH)], wsem[s])
    for s in range(min(_NBUF, nch)):
      w[s].wait()

  return k(ids3, table)


def _tc_body(x_ref, cw_ref, w_ref, out_ref, carry_ref):
  j = pl.program_id(0)

  @pl.when(j % _BPB == 0)
  def _():
    carry_ref[...] = jnp.zeros_like(carry_ref)

  x4 = x_ref[...]
  prev = carry_ref[...]
  cw = cw_ref[...]
  conv = x4 * cw[:, _K - 1, :][:, None, :]
  for s in range(1, _K):
    shifted = jnp.concatenate(
        [prev[:, 8 - s:, :], x4[:, :_TL - s, :]], axis=1)
    conv = conv + shifted * cw[:, _K - 1 - s, :][:, None, :]
  carry_ref[...] = x4[:, _TL - 8:, :]
  y4 = (conv * jax.nn.sigmoid(conv) * x4).astype(jnp.bfloat16)
  y = jnp.concatenate([y4[0], y4[1], y4[2], y4[3]], axis=1)
  out_ref[...] = jnp.dot(y, w_ref[...], preferred_element_type=jnp.float32)


def _tc_call(x4, cw4, w_bf16, bl):
  return pl.pallas_call(
      _tc_body,
      grid=(bl // _TL,),
      in_specs=[
          pl.BlockSpec((_H, _TL, _D), lambda j: (0, j, 0)),
          pl.BlockSpec((_H, _K, _D), lambda j: (0, 0, 0)),
          pl.BlockSpec((_HD, _DM), lambda j: (0, 0)),
      ],
      out_specs=pl.BlockSpec((_TL, _DM), lambda j: (j, 0)),
      out_shape=jax.ShapeDtypeStruct((bl, _DM), jnp.float32),
      scratch_shapes=[pltpu.VMEM((_H, 8, _D), jnp.float32)],
      compiler_params=pltpu.CompilerParams(
          dimension_semantics=("arbitrary",)),
  )(x4, cw4, w_bf16)


def kernel(input_ids, emb_table, conv_w, w_out):
  offsets = jnp.array(np.cumsum([0] + _LIST_OF_N[:-1]), dtype=input_ids.dtype)
  shifted = (input_ids + offsets[None, None, :]).transpose(2, 0, 1)
  shifted = shifted.reshape(_H, _BL)
  cw4 = conv_w.reshape(_K, _H, _D).transpose(1, 0, 2)
  w16 = w_out.astype(jnp.bfloat16)
  bl = _BL // _SPLIT
  nch = _H * bl // _NW // _CH
  outs = []
  for p in range(_SPLIT):
    ids3 = shifted[:, p * bl:(p + 1) * bl].reshape(_NW, nch, _CH)
    x4 = _gather_sc(ids3, emb_table, bl)
    outs.append(_tc_call(x4, cw4, w16, bl))
  out = outs[0] if _SPLIT == 1 else jnp.concatenate(outs, axis=0)
  return out.reshape(_B, _L, _DM)
